# baseline (speedup 1.0000x reference)
import jax
import jax.numpy as jnp
from jax.experimental import pallas as pl
from jax.experimental.pallas import tpu as pltpu

_N = 1 << 20
_B = 1 << 10
_C = 256
_W = 256
_NS = 4


def _wht_kernel(xr_ref, xi_ref, or_ref, oi_ref, h_ref,
                ur_ref, ui_ref, ar_ref, ai_ref):
    j = pl.program_id(0)

    @pl.when(j == 0)
    def _gen_h():
        r = jax.lax.broadcasted_iota(jnp.int32, (_C, _C), 0)
        c = jax.lax.broadcasted_iota(jnp.int32, (_C, _C), 1)
        parity = jax.lax.population_count(r & c) & 1
        bits = (0x3F80 | (parity << 15)).astype(jnp.uint16)
        h_ref[...] = jax.lax.bitcast_convert_type(bits, jnp.bfloat16)

    @pl.when(j < _NS)
    def _stage1():
        h = h_ref[...]
        row = pl.ds(j * _W, _W)
        for x_ref, u_ref in ((xr_ref, ur_ref), (xi_ref, ui_ref)):
            x = x_ref[...]
            c0 = x[:, 0 * _C:1 * _C]
            c1 = x[:, 1 * _C:2 * _C]
            c2 = x[:, 2 * _C:3 * _C]
            c3 = x[:, 3 * _C:4 * _C]
            p0, p1 = c0 + c1, c0 - c1
            p2, p3 = c2 + c3, c2 - c3
            s = (p0 + p2, p1 + p3, p0 - p2, p1 - p3)
            for b in range(4):
                u_ref[row, b * _C:(b + 1) * _C] = jax.lax.dot(
                    s[b].astype(jnp.bfloat16), h,
                    preferred_element_type=jnp.float32).astype(jnp.bfloat16)

    @pl.when(j == _NS)
    def _level1():
        for u_ref, a_ref in ((ur_ref, ar_ref), (ui_ref, ai_ref)):
            u0 = u_ref[0 * _C:1 * _C, :]
            u1 = u_ref[1 * _C:2 * _C, :]
            u2 = u_ref[2 * _C:3 * _C, :]
            u3 = u_ref[3 * _C:4 * _C, :]
            a_ref[0 * _C:1 * _C, :] = u0 + u1
            a_ref[1 * _C:2 * _C, :] = u0 - u1
            a_ref[2 * _C:3 * _C, :] = u2 + u3
            a_ref[3 * _C:4 * _C, :] = u2 - u3

    @pl.when(j >= _NS)
    def _stage2():
        h = h_ref[...]
        b = j - _NS
        sign = jnp.where(b < 2, 1.0, -1.0).astype(jnp.bfloat16)
        lo_row = pl.ds((b % 2) * _C, _C)
        hi_row = pl.ds((b % 2) * _C + 2 * _C, _C)
        for a_ref, o_ref in ((ar_ref, or_ref), (ai_ref, oi_ref)):
            s = a_ref[lo_row, :] + sign * a_ref[hi_row, :]
            o_ref[...] = jax.lax.dot(h, s, preferred_element_type=jnp.float32)


def kernel(x_real, x_imag):
    yr, yi = pl.pallas_call(
        _wht_kernel,
        grid=(2 * _NS,),
        in_specs=[
            pl.BlockSpec((_W, _B), lambda j: (jnp.minimum(j, _NS - 1), 0)),
            pl.BlockSpec((_W, _B), lambda j: (jnp.minimum(j, _NS - 1), 0)),
        ],
        out_specs=(
            pl.BlockSpec((_W, _B), lambda j: (jnp.maximum(j - _NS, 0), 0)),
            pl.BlockSpec((_W, _B), lambda j: (jnp.maximum(j - _NS, 0), 0)),
        ),
        out_shape=(jax.ShapeDtypeStruct((_B, _B), jnp.float32),
                   jax.ShapeDtypeStruct((_B, _B), jnp.float32)),
        scratch_shapes=[
            pltpu.VMEM((_C, _C), jnp.bfloat16),
            pltpu.VMEM((_B, _B), jnp.bfloat16),
            pltpu.VMEM((_B, _B), jnp.bfloat16),
            pltpu.VMEM((_B, _B), jnp.bfloat16),
            pltpu.VMEM((_B, _B), jnp.bfloat16),
        ],
    )(x_real.reshape(_B, _B), x_imag.reshape(_B, _B))
    return yr.reshape(_N), yi.reshape(_N)


# speedup vs baseline: 1.0020x; 1.0020x over previous
import jax
import jax.numpy as jnp
from jax.experimental import pallas as pl
from jax.experimental.pallas import tpu as pltpu

_N = 1 << 20
_B = 1 << 10
_C = 256
_W = 256
_NS = 4


def _wht_kernel(xr_ref, xi_ref, or_ref, oi_ref, h_ref,
                ur_ref, ui_ref, ar_ref, ai_ref):
    j = pl.program_id(0)

    @pl.when(j == 0)
    def _gen_h():
        r = jax.lax.broadcasted_iota(jnp.int32, (_C, _C), 0)
        c = jax.lax.broadcasted_iota(jnp.int32, (_C, _C), 1)
        parity = jax.lax.population_count(r & c) & 1
        bits = (0x3F80 | (parity << 15)).astype(jnp.uint16)
        h_ref[...] = jax.lax.bitcast_convert_type(bits, jnp.bfloat16)

    @pl.when(j < _NS)
    def _stage1():
        h = h_ref[...]
        row = pl.ds(j * _W, _W)
        for x_ref, u_ref in ((xr_ref, ur_ref), (xi_ref, ui_ref)):
            x = x_ref[...]
            c0 = x[:, 0 * _C:1 * _C]
            c1 = x[:, 1 * _C:2 * _C]
            c2 = x[:, 2 * _C:3 * _C]
            c3 = x[:, 3 * _C:4 * _C]
            p0, p1 = c0 + c1, c0 - c1
            p2, p3 = c2 + c3, c2 - c3
            s = (p0 + p2, p1 + p3, p0 - p2, p1 - p3)
            for b in range(4):
                u_ref[row, b * _C:(b + 1) * _C] = jax.lax.dot(
                    s[b].astype(jnp.bfloat16), h,
                    preferred_element_type=jnp.float32).astype(jnp.bfloat16)

    @pl.when(j == _NS - 1)
    def _level1():
        for u_ref, a_ref in ((ur_ref, ar_ref), (ui_ref, ai_ref)):
            u0 = u_ref[0 * _C:1 * _C, :]
            u1 = u_ref[1 * _C:2 * _C, :]
            u2 = u_ref[2 * _C:3 * _C, :]
            u3 = u_ref[3 * _C:4 * _C, :]
            a_ref[0 * _C:1 * _C, :] = u0 + u1
            a_ref[1 * _C:2 * _C, :] = u0 - u1
            a_ref[2 * _C:3 * _C, :] = u2 + u3
            a_ref[3 * _C:4 * _C, :] = u2 - u3

    @pl.when(j >= _NS - 1)
    def _stage2():
        h = h_ref[...]
        b = j - (_NS - 1)
        sign = jnp.where(b < 2, 1.0, -1.0).astype(jnp.bfloat16)
        lo_row = pl.ds((b % 2) * _C, _C)
        hi_row = pl.ds((b % 2) * _C + 2 * _C, _C)
        for a_ref, o_ref in ((ar_ref, or_ref), (ai_ref, oi_ref)):
            s = a_ref[lo_row, :] + sign * a_ref[hi_row, :]
            o_ref[...] = jax.lax.dot(h, s, preferred_element_type=jnp.float32)


def kernel(x_real, x_imag):
    yr, yi = pl.pallas_call(
        _wht_kernel,
        grid=(2 * _NS - 1,),
        in_specs=[
            pl.BlockSpec((_W, _B), lambda j: (jnp.minimum(j, _NS - 1), 0)),
            pl.BlockSpec((_W, _B), lambda j: (jnp.minimum(j, _NS - 1), 0)),
        ],
        out_specs=(
            pl.BlockSpec((_W, _B), lambda j: (jnp.maximum(j - (_NS - 1), 0), 0)),
            pl.BlockSpec((_W, _B), lambda j: (jnp.maximum(j - (_NS - 1), 0), 0)),
        ),
        out_shape=(jax.ShapeDtypeStruct((_B, _B), jnp.float32),
                   jax.ShapeDtypeStruct((_B, _B), jnp.float32)),
        scratch_shapes=[
            pltpu.VMEM((_C, _C), jnp.bfloat16),
            pltpu.VMEM((_B, _B), jnp.bfloat16),
            pltpu.VMEM((_B, _B), jnp.bfloat16),
            pltpu.VMEM((_B, _B), jnp.bfloat16),
            pltpu.VMEM((_B, _B), jnp.bfloat16),
        ],
    )(x_real.reshape(_B, _B), x_imag.reshape(_B, _B))
    return yr.reshape(_N), yi.reshape(_N)


# speedup vs baseline: 1.0493x; 1.0472x over previous
import jax
import jax.numpy as jnp
from jax.experimental import pallas as pl
from jax.experimental.pallas import tpu as pltpu

_N = 1 << 20
_B = 1 << 10
_C = 256
_W = 512
_NS = 2


def _wht_kernel(xr_ref, xi_ref, or_ref, oi_ref, h_ref,
                ur_ref, ui_ref, ar_ref, ai_ref):
    j = pl.program_id(0)

    @pl.when(j == 0)
    def _gen_h():
        r = jax.lax.broadcasted_iota(jnp.int32, (_C, _C), 0)
        c = jax.lax.broadcasted_iota(jnp.int32, (_C, _C), 1)
        parity = jax.lax.population_count(r & c) & 1
        bits = (0x3F80 | (parity << 15)).astype(jnp.uint16)
        h_ref[...] = jax.lax.bitcast_convert_type(bits, jnp.bfloat16)

    @pl.when(j < _NS)
    def _stage1():
        h = h_ref[...]
        row = pl.ds(j * _W, _W)
        for x_ref, u_ref in ((xr_ref, ur_ref), (xi_ref, ui_ref)):
            x = x_ref[...]
            c0 = x[:, 0 * _C:1 * _C]
            c1 = x[:, 1 * _C:2 * _C]
            c2 = x[:, 2 * _C:3 * _C]
            c3 = x[:, 3 * _C:4 * _C]
            p0, p1 = c0 + c1, c0 - c1
            p2, p3 = c2 + c3, c2 - c3
            s = (p0 + p2, p1 + p3, p0 - p2, p1 - p3)
            for b in range(4):
                u_ref[row, b * _C:(b + 1) * _C] = jax.lax.dot(
                    s[b].astype(jnp.bfloat16), h,
                    preferred_element_type=jnp.float32).astype(jnp.bfloat16)

    @pl.when(j == _NS - 1)
    def _level1():
        for u_ref, a_ref in ((ur_ref, ar_ref), (ui_ref, ai_ref)):
            u0 = u_ref[0 * _C:1 * _C, :]
            u1 = u_ref[1 * _C:2 * _C, :]
            u2 = u_ref[2 * _C:3 * _C, :]
            u3 = u_ref[3 * _C:4 * _C, :]
            a_ref[0 * _C:1 * _C, :] = u0 + u1
            a_ref[1 * _C:2 * _C, :] = u0 - u1
            a_ref[2 * _C:3 * _C, :] = u2 + u3
            a_ref[3 * _C:4 * _C, :] = u2 - u3

    @pl.when(j >= _NS - 1)
    def _stage2():
        h = h_ref[...]
        b = j - (_NS - 1)
        for gg in range(_W // _C):
            g = b * (_W // _C) + gg
            sign = jnp.where(g < 2, 1.0, -1.0).astype(jnp.bfloat16)
            lo_row = pl.ds((g % 2) * _C, _C)
            hi_row = pl.ds((g % 2) * _C + 2 * _C, _C)
            for a_ref, o_ref in ((ar_ref, or_ref), (ai_ref, oi_ref)):
                s = a_ref[lo_row, :] + sign * a_ref[hi_row, :]
                o_ref[gg * _C:(gg + 1) * _C, :] = jax.lax.dot(
                    h, s, preferred_element_type=jnp.float32)


def kernel(x_real, x_imag):
    yr, yi = pl.pallas_call(
        _wht_kernel,
        grid=(2 * _NS - 1,),
        in_specs=[
            pl.BlockSpec((_W, _B), lambda j: (jnp.minimum(j, _NS - 1), 0)),
            pl.BlockSpec((_W, _B), lambda j: (jnp.minimum(j, _NS - 1), 0)),
        ],
        out_specs=(
            pl.BlockSpec((_W, _B), lambda j: (jnp.maximum(j - (_NS - 1), 0), 0)),
            pl.BlockSpec((_W, _B), lambda j: (jnp.maximum(j - (_NS - 1), 0), 0)),
        ),
        out_shape=(jax.ShapeDtypeStruct((_B, _B), jnp.float32),
                   jax.ShapeDtypeStruct((_B, _B), jnp.float32)),
        scratch_shapes=[
            pltpu.VMEM((_C, _C), jnp.bfloat16),
            pltpu.VMEM((_B, _B), jnp.bfloat16),
            pltpu.VMEM((_B, _B), jnp.bfloat16),
            pltpu.VMEM((_B, _B), jnp.bfloat16),
            pltpu.VMEM((_B, _B), jnp.bfloat16),
        ],
    )(x_real.reshape(_B, _B), x_imag.reshape(_B, _B))
    return yr.reshape(_N), yi.reshape(_N)
